# initial kernel scaffold (unmeasured)
import jax
import jax.numpy as jnp
from jax import lax
from jax.experimental import pallas as pl
from jax.experimental.pallas import tpu as pltpu


def kernel(
    x,
):
    def body(*refs):
        pass

    out_shape = jax.ShapeDtypeStruct(..., jnp.float32)
    return pl.pallas_call(body, out_shape=out_shape)(...)


# baseline (device time: 1067913 ns/iter reference)
import jax
import jax.numpy as jnp
from jax import lax
from jax.experimental import pallas as pl
from jax.experimental.pallas import tpu as pltpu

XS = 2


def kernel(x):
    m, n = x.shape
    ncol = n // XS
    out_m = m * XS

    def body(x_ref, out_ref, local_sem, send_sem, recv_sem):
        my_x = lax.axis_index("x")
        my_y = lax.axis_index("y")
        my_z = lax.axis_index("z")
        other_x = 1 - my_x

        barrier_sem = pltpu.get_barrier_semaphore()
        pl.semaphore_signal(
            barrier_sem,
            inc=1,
            device_id=(other_x, my_y, my_z),
            device_id_type=pl.DeviceIdType.MESH,
        )
        pl.semaphore_wait(barrier_sem, 1)

        local = pltpu.make_async_copy(
            x_ref.at[:, pl.ds(my_x * ncol, ncol)],
            out_ref.at[pl.ds(my_x * m, m), :],
            local_sem,
        )
        local.start()

        rdma = pltpu.make_async_remote_copy(
            src_ref=x_ref.at[:, pl.ds(other_x * ncol, ncol)],
            dst_ref=out_ref.at[pl.ds(my_x * m, m), :],
            send_sem=send_sem,
            recv_sem=recv_sem,
            device_id=(other_x, my_y, my_z),
            device_id_type=pl.DeviceIdType.MESH,
        )
        rdma.start()

        local.wait()
        rdma.wait()

    return pl.pallas_call(
        body,
        out_shape=jax.ShapeDtypeStruct((out_m, ncol), x.dtype),
        in_specs=[pl.BlockSpec(memory_space=pl.ANY)],
        out_specs=pl.BlockSpec(memory_space=pl.ANY),
        scratch_shapes=[
            pltpu.SemaphoreType.DMA,
            pltpu.SemaphoreType.DMA,
            pltpu.SemaphoreType.DMA,
        ],
        compiler_params=pltpu.CompilerParams(collective_id=0),
    )(x)


# device time: 1067171 ns/iter; 1.0007x vs baseline; 1.0007x over previous
import jax
import jax.numpy as jnp
from jax import lax
from jax.experimental import pallas as pl
from jax.experimental.pallas import tpu as pltpu

XS = 2
CHUNKS = 8
LOCAL_CHUNKS = 4


def kernel(x):
    m, n = x.shape
    ncol = n // XS
    out_m = m * XS

    def body(x_ref, out_ref, local_sems, send_sems, recv_sems):
        my_x = lax.axis_index("x")
        my_y = lax.axis_index("y")
        my_z = lax.axis_index("z")
        other_x = 1 - my_x

        barrier_sem = pltpu.get_barrier_semaphore()
        pl.semaphore_signal(
            barrier_sem,
            inc=1,
            device_id=(other_x, my_y, my_z),
            device_id_type=pl.DeviceIdType.MESH,
        )
        pl.semaphore_wait(barrier_sem, 1)

        rows = m // CHUNKS
        rdmas = []
        for i in range(CHUNKS):
            rdma = pltpu.make_async_remote_copy(
                src_ref=x_ref.at[
                    pl.ds(i * rows, rows), pl.ds(other_x * ncol, ncol)
                ],
                dst_ref=out_ref.at[pl.ds(my_x * m + i * rows, rows), :],
                send_sem=send_sems.at[i],
                recv_sem=recv_sems.at[i],
                device_id=(other_x, my_y, my_z),
                device_id_type=pl.DeviceIdType.MESH,
            )
            rdma.start()
            rdmas.append(rdma)

        lrows = m // LOCAL_CHUNKS
        locals_ = []
        for i in range(LOCAL_CHUNKS):
            local = pltpu.make_async_copy(
                x_ref.at[pl.ds(i * lrows, lrows), pl.ds(my_x * ncol, ncol)],
                out_ref.at[pl.ds(my_x * m + i * lrows, lrows), :],
                local_sems.at[i],
            )
            local.start()
            locals_.append(local)

        for local in locals_:
            local.wait()
        for rdma in rdmas:
            rdma.wait()

    return pl.pallas_call(
        body,
        out_shape=jax.ShapeDtypeStruct((out_m, ncol), x.dtype),
        in_specs=[pl.BlockSpec(memory_space=pl.ANY)],
        out_specs=pl.BlockSpec(memory_space=pl.ANY),
        scratch_shapes=[
            pltpu.SemaphoreType.DMA((LOCAL_CHUNKS,)),
            pltpu.SemaphoreType.DMA((CHUNKS,)),
            pltpu.SemaphoreType.DMA((CHUNKS,)),
        ],
        compiler_params=pltpu.CompilerParams(collective_id=0),
    )(x)


# device time: 409271 ns/iter; 2.6093x vs baseline; 2.6075x over previous
import jax
import jax.numpy as jnp
from jax import lax
from jax.experimental import pallas as pl
from jax.experimental.pallas import tpu as pltpu

XS = 2
CHUNK_ROWS = 1024


def kernel(x):
    m, n = x.shape
    ncol = n // XS
    out_m = m * XS

    def body(x_ref, out_ref, vmem_ref, read_sems, write_sems, send_sem, recv_sem):
        my_x = lax.axis_index("x")
        my_y = lax.axis_index("y")
        my_z = lax.axis_index("z")
        other_x = 1 - my_x

        barrier_sem = pltpu.get_barrier_semaphore()
        pl.semaphore_signal(
            barrier_sem,
            inc=1,
            device_id=(other_x, my_y, my_z),
            device_id_type=pl.DeviceIdType.MESH,
        )
        pl.semaphore_wait(barrier_sem, 1)

        rdma = pltpu.make_async_remote_copy(
            src_ref=x_ref.at[:, pl.ds(other_x * ncol, ncol)],
            dst_ref=out_ref.at[pl.ds(my_x * m, m), :],
            send_sem=send_sem,
            recv_sem=recv_sem,
            device_id=(other_x, my_y, my_z),
            device_id_type=pl.DeviceIdType.MESH,
        )
        rdma.start()

        nchunks = m // CHUNK_ROWS
        reads = []
        writes = []
        for i in range(nchunks):
            slot = i % 2
            if i >= 2:
                writes[i - 2].wait()
            rd = pltpu.make_async_copy(
                x_ref.at[
                    pl.ds(i * CHUNK_ROWS, CHUNK_ROWS),
                    pl.ds(my_x * ncol, ncol),
                ],
                vmem_ref.at[slot],
                read_sems.at[slot],
            )
            rd.start()
            reads.append(rd)
            rd.wait()
            wr = pltpu.make_async_copy(
                vmem_ref.at[slot],
                out_ref.at[pl.ds(my_x * m + i * CHUNK_ROWS, CHUNK_ROWS), :],
                write_sems.at[slot],
            )
            wr.start()
            writes.append(wr)

        writes[nchunks - 2].wait()
        writes[nchunks - 1].wait()
        rdma.wait()

    return pl.pallas_call(
        body,
        out_shape=jax.ShapeDtypeStruct((out_m, ncol), x.dtype),
        in_specs=[pl.BlockSpec(memory_space=pl.ANY)],
        out_specs=pl.BlockSpec(memory_space=pl.ANY),
        scratch_shapes=[
            pltpu.VMEM((2, CHUNK_ROWS, ncol), x.dtype),
            pltpu.SemaphoreType.DMA((2,)),
            pltpu.SemaphoreType.DMA((2,)),
            pltpu.SemaphoreType.DMA,
            pltpu.SemaphoreType.DMA,
        ],
        compiler_params=pltpu.CompilerParams(collective_id=0),
    )(x)
